# Initial kernel scaffold; baseline (speedup 1.0000x reference)
import jax
import jax.numpy as jnp
from jax.experimental import pallas as pl


def kernel(x, edge_index, edge_attr, k, params):
    raise NotImplementedError("write your pallas kernel here")


# speedup vs baseline: 1.0015x; 1.0015x over previous
import functools

import jax
import jax.numpy as jnp
from jax import lax
from jax.experimental import pallas as pl
from jax.experimental.pallas import tpu as pltpu

N = 10000
NUM_CONV = 6
K_HOPS = 3
_PAD = 10240


def _instance_norm(x, eps=1e-5):
    mean = jnp.mean(x, axis=0, keepdims=True)
    var = jnp.var(x, axis=0, keepdims=True)
    return (x - mean) / jnp.sqrt(var + eps)


def _gcn_norm(edge_index, edge_weight, n):
    row, col = edge_index[0], edge_index[1]
    deg = jnp.zeros((n,), edge_weight.dtype).at[col].add(edge_weight)
    dinv = jnp.where(deg > 0, 1.0 / jnp.sqrt(jnp.maximum(deg, 1e-12)), 0.0)
    return dinv[row] * edge_weight * dinv[col]


def _tagconv(x, edge_index, norm_w, tag_ws, tag_b, n):
    out = x @ tag_ws[0]
    h = x
    row, col = edge_index[0], edge_index[1]
    for kk in range(1, len(tag_ws)):
        msg = h[row] * norm_w[:, None]
        h = jnp.zeros((n, h.shape[1]), x.dtype).at[col].add(msg)
        out = out + h @ tag_ws[kk]
    return out + tag_b


def _forward(x, edge_index, edge_attr, params):
    if x.ndim == 1:
        x = x[:, None]
    n = x.shape[0]
    norm_w = _gcn_norm(edge_index, edge_attr, n)
    for i in range(NUM_CONV):
        p = params[i]
        x = _instance_norm(x)
        x = _tagconv(x, edge_index, norm_w, p["tag_ws"], p["tag_b"], n)
        x = jax.nn.relu(x)
        for (w, b) in p["fc"]:
            x = jax.nn.relu(x @ w + b)
    return x


def _topk_body(k_ref, s_ref, o_ref):
    kk = k_ref[0]
    s = s_ref[...]

    def t_step(i, t):
        cand = t | (jnp.int32(1) << (30 - i))
        cnt = jnp.sum((s >= cand).astype(jnp.int32))
        return jnp.where(cnt >= kk, cand, t)

    T = lax.fori_loop(0, 31, t_step, jnp.int32(0))

    cnt_gt = jnp.sum((s > T).astype(jnp.int32))
    need = kk - cnt_gt
    eq = s == T
    idx = lax.broadcasted_iota(jnp.int32, (80, 128), 0) * 128 + lax.broadcasted_iota(
        jnp.int32, (80, 128), 1
    )

    def m_step(i, m):
        cand = m | (jnp.int32(1) << (13 - i))
        cnt = jnp.sum((eq & (idx < cand)).astype(jnp.int32))
        return jnp.where(cnt < need, cand, m)

    m = lax.fori_loop(0, 14, m_step, jnp.int32(0))
    mask = (s > T) | (eq & (idx <= m))
    o_ref[...] = mask.astype(jnp.float32)


def _topk_mask(scores, k):
    s = scores + 0.0
    s = jnp.concatenate([s, jnp.full((_PAD - N,), -1.0, jnp.float32)])
    sbits = lax.bitcast_convert_type(s, jnp.int32).reshape(80, 128)
    karr = jnp.asarray(k, jnp.int32).reshape(1)
    out = pl.pallas_call(
        _topk_body,
        out_shape=jax.ShapeDtypeStruct((80, 128), jnp.float32),
        in_specs=[
            pl.BlockSpec(memory_space=pltpu.SMEM),
            pl.BlockSpec(memory_space=pltpu.VMEM),
        ],
        out_specs=pl.BlockSpec(memory_space=pltpu.VMEM),
    )(karr, sbits)
    return out.reshape(-1)[:N]


def kernel(x, edge_index, edge_attr, k, params):
    scores = _forward(x, edge_index, edge_attr, params).squeeze()
    return (_topk_mask(scores, k), edge_attr)


# speedup vs baseline: 4.9455x; 4.9379x over previous
import dataclasses
import functools

import jax
import jax.numpy as jnp
from jax import lax
from jax.experimental import pallas as pl
from jax.experimental.pallas import tpu as pltpu
from jax.experimental.pallas import tpu_sc as plsc

N = 10000
NPAD = 10240
E = 320000
EPAD = 327680
NBLK = 160
DIM = 128
NSUB = 16
RPS = NPAD // NSUB
RPS2 = N // NSUB

@functools.cache
def _mesh():
    return plsc.VectorSubcoreMesh(core_axis_name="core", subcore_axis_name="subcore")


def _sc_params(layout_passes=True):
    cp = pltpu.CompilerParams()
    if not layout_passes and "needs_layout_passes" in pltpu.CompilerParams.__dataclass_fields__:
        cp = dataclasses.replace(cp, needs_layout_passes=False)
    return cp


def _deg_body(rc, eab, dg, acc1, cb, eb, t1):
    c = lax.axis_index("core")
    s = lax.axis_index("subcore")

    @pl.loop(0, RPS, step=16)
    def _(r):
        t1[pl.ds(r, 16)] = jnp.zeros((16,), jnp.float32)

    pltpu.sync_copy(t1, acc1.at[pl.ds(s * RPS, RPS)])
    plsc.subcore_barrier()

    w = c * NSUB + s

    @pl.loop(0, 5)
    def _(bi):
        b = w * 5 + bi
        pltpu.sync_copy(rc.at[b, 1], cb)
        pltpu.sync_copy(eab.at[b], eb)

        @pl.loop(0, 16)
        def _(j):
            pltpu.sync_copy(eb.at[j], acc1.at[cb.at[j]], add=True)

    plsc.subcore_barrier()
    pltpu.sync_copy(acc1.at[pl.ds(s * RPS, RPS)], t1)
    pltpu.sync_copy(t1, dg.at[c, pl.ds(s * RPS, RPS)])


def _deg_call(rc, eab):
    f = pl.kernel(
        _deg_body,
        out_type=jax.ShapeDtypeStruct((2, NPAD), jnp.float32),
        mesh=_mesh(),
        scratch_types=[
            pltpu.VMEM_SHARED((NPAD,), jnp.float32),
            pltpu.VMEM((16, 128), jnp.int32),
            pltpu.VMEM((16, 128), jnp.float32),
            pltpu.VMEM((RPS,), jnp.float32),
        ],
        compiler_params=_sc_params(),
    )
    return f(rc, eab)


def _nsc_body(rc, eab, dinv, xn0, wn, hsc, wn_sh, acc1, tv, dv, rcb, eb, mb, t1):
    c = lax.axis_index("core")
    s = lax.axis_index("subcore")

    pltpu.sync_copy(dinv, dv)

    @pl.loop(0, RPS, step=16)
    def _(r):
        t1[pl.ds(r, 16)] = jnp.zeros((16,), jnp.float32)

    pltpu.sync_copy(t1, acc1.at[pl.ds(s * RPS, RPS)])

    @pl.loop(0, 10)
    def _(bi):
        b = s * 10 + bi
        pltpu.sync_copy(rc.at[b], rcb)
        pltpu.sync_copy(eab.at[b], eb)

        @pl.loop(0, 16)
        def _(j):
            for l in range(8):
                sl = pl.ds(l * 16, 16)
                dr = plsc.load_gather(dv, [rcb[0, j, sl]])
                dc = plsc.load_gather(dv, [rcb[1, j, sl]])
                mb[j, sl] = dr * eb[j, sl] * dc

        pltpu.sync_copy(mb, wn_sh.at[b])

        @pl.when(c == 0)
        def _():
            pltpu.sync_copy(mb, wn.at[b])

    pltpu.sync_copy(xn0, tv)
    plsc.subcore_barrier()

    for hop in range(3):
        @pl.loop(0, 10)
        def _(bi, hop=hop):
            b = s * 10 + bi
            pltpu.sync_copy(rc.at[b], rcb)
            pltpu.sync_copy(wn_sh.at[b], eb)

            @pl.loop(0, 16)
            def _(j):
                for l in range(8):
                    sl = pl.ds(l * 16, 16)
                    hv = plsc.load_gather(tv, [rcb[0, j, sl]])
                    mb[j, sl] = hv * eb[j, sl]
                pltpu.sync_copy(mb.at[j], acc1.at[rcb.at[1, j]], add=True)

        plsc.subcore_barrier()
        pltpu.sync_copy(acc1, tv)

        @pl.when(c == 0)
        def _(hop=hop):
            pltpu.sync_copy(tv.at[pl.ds(s * RPS, RPS)],
                            hsc.at[hop + 1, pl.ds(s * RPS, RPS)])

        plsc.subcore_barrier()
        pltpu.sync_copy(t1, acc1.at[pl.ds(s * RPS, RPS)])
        plsc.subcore_barrier()

    @pl.when(c == 0)
    def _():
        pltpu.sync_copy(xn0.at[pl.ds(s * RPS, RPS)], t1)
        pltpu.sync_copy(t1, hsc.at[0, pl.ds(s * RPS, RPS)])


def _nsc_call(rc, eab, dinv, xn0):
    f = pl.kernel(
        _nsc_body,
        out_type=(
            jax.ShapeDtypeStruct((NBLK, 16, 128), jnp.float32),
            jax.ShapeDtypeStruct((4, NPAD), jnp.float32),
        ),
        mesh=_mesh(),
        scratch_types=[
            pltpu.VMEM_SHARED((NBLK, 16, 128), jnp.float32),
            pltpu.VMEM_SHARED((NPAD,), jnp.float32),
            pltpu.VMEM((NPAD,), jnp.float32),
            pltpu.VMEM((NPAD,), jnp.float32),
            pltpu.VMEM((2, 16, 128), jnp.int32),
            pltpu.VMEM((16, 128), jnp.float32),
            pltpu.VMEM((16, 128), jnp.float32),
            pltpu.VMEM((RPS,), jnp.float32),
        ],
        compiler_params=_sc_params(layout_passes=False),
    )
    return f(rc, eab, dinv, xn0)


def _hop_body(two, *args):
    if two:
        srcA, srcB, rc, wn, out, acc, rcb, wv, gbuf, tio, zbuf = args
        srcs = [srcA, srcB]
    else:
        srcA, rc, wn, out, acc, rcb, wv, gbuf, tio, zbuf = args
        srcs = [srcA]
    c = lax.axis_index("core")
    s = lax.axis_index("subcore")

    @pl.loop(0, 64)
    def _(r):
        for q in range(8):
            zbuf[r, pl.ds(q * 16, 16)] = jnp.zeros((16,), jnp.float32)

    for t in range(10):
        pltpu.sync_copy(zbuf, acc.at[pl.ds(s * RPS + t * 64, 64)])
    plsc.subcore_barrier()

    def chunk(b, j):
        pltpu.sync_copy(srcs[0].at[rcb.at[0, j]], gbuf)
        if two:
            pltpu.sync_copy(srcs[1].at[rcb.at[0, j]], gbuf, add=True)
        pltpu.sync_copy(wn.at[b, j], wv.at[pl.ds(0, 128)])

        @pl.loop(0, 128)
        def _(e):
            ww = wv[pl.ds(e, 16)][0]
            for q in range(8):
                gbuf[e, pl.ds(q * 16, 16)] = gbuf[e, pl.ds(q * 16, 16)] * ww

        pltpu.sync_copy(gbuf, acc.at[rcb.at[1, j]], add=True)

    for ci in range(2):
        @pl.when(c == ci)
        def _(ci=ci):
            @pl.loop(0, 5)
            def _(bi, ci=ci):
                b = ci * 80 + s * 5 + bi
                pltpu.sync_copy(rc.at[b], rcb)

                @pl.loop(0, 16)
                def _(j, b=b):
                    chunk(b, j)

    plsc.subcore_barrier()
    for t in range(10):
        sl = pl.ds(s * RPS + t * 64, 64)
        pltpu.sync_copy(acc.at[sl], tio)
        pltpu.sync_copy(tio, out.at[c, sl])


def _hop_call(srcA, srcB, rc, wn):
    two = srcB is not None
    f = pl.kernel(
        functools.partial(_hop_body, two),
        out_type=jax.ShapeDtypeStruct((2, NPAD, 128), jnp.float32),
        mesh=_mesh(),
        scratch_types=[
            pltpu.VMEM_SHARED((NPAD, 128), jnp.float32),
            pltpu.VMEM((2, 16, 128), jnp.int32),
            pltpu.VMEM((144,), jnp.float32),
            pltpu.VMEM((128, 128), jnp.float32),
            pltpu.VMEM((64, 128), jnp.float32),
            pltpu.VMEM((64, 128), jnp.float32),
        ],
        compiler_params=_sc_params(),
    )
    if two:
        return f(srcA, srcB, rc, wn)
    return f(srcA, rc, wn)


def _hops_call(xn, rc, wn):
    p1 = _hop_call(xn, None, rc, wn)
    p2 = _hop_call(p1[0], p1[1], rc, wn)
    p3 = _hop_call(p2[0], p2[1], rc, wn)
    return jnp.stack([p1, p2, p3])


def _prep_body(x_ref, da_ref, db_ref, dinv_ref, xn_ref):
    x = x_ref[...]
    deg = da_ref[...] + db_ref[...]
    idx = lax.broadcasted_iota(jnp.int32, (80, 128), 0) * 128 + \
        lax.broadcasted_iota(jnp.int32, (80, 128), 1)
    mask = idx < N
    dinv_ref[...] = jnp.where(deg > 0,
                              1.0 / jnp.sqrt(jnp.maximum(deg, 1e-12)), 0.0)
    mu = jnp.sum(jnp.where(mask, x, 0.0)) / N
    var = jnp.sum(jnp.where(mask, (x - mu) ** 2, 0.0)) / N
    xn_ref[...] = (x - mu) / jnp.sqrt(var + 1e-5)


def _prep_call(xpad, dg):
    out = pl.pallas_call(
        _prep_body,
        out_shape=(jax.ShapeDtypeStruct((80, 128), jnp.float32),
                   jax.ShapeDtypeStruct((80, 128), jnp.float32)),
    )(xpad.reshape(80, 128), dg[0].reshape(80, 128), dg[1].reshape(80, 128))
    return out[0].reshape(NPAD), out[1].reshape(NPAD)


def _norm_body(y_ref, xn_ref):
    y = y_ref[...]
    rmask = lax.broadcasted_iota(jnp.int32, (NPAD, 1), 0) < N
    ym = jnp.where(rmask, y, 0.0)
    mu = jnp.sum(ym, axis=0, keepdims=True) / N
    var = jnp.sum(jnp.where(rmask, (y - mu) ** 2, 0.0), axis=0, keepdims=True) / N
    xn_ref[...] = (y - mu) / jnp.sqrt(var + 1e-5)


def _norm_call(y):
    return pl.pallas_call(
        _norm_body,
        out_shape=jax.ShapeDtypeStruct((NPAD, 128), jnp.float32),
    )(y)


def _dense0_body(h_ref, w0_ref, tb_ref, fw_ref, fb_ref, y_ref):
    y = jnp.dot(h_ref[...], w0_ref[...], preferred_element_type=jnp.float32, precision=lax.Precision.HIGHEST)
    y = y + tb_ref[...]
    y = jnp.maximum(y, 0.0)
    for f in range(5):
        y = jnp.dot(y, fw_ref[f], preferred_element_type=jnp.float32, precision=lax.Precision.HIGHEST) + fb_ref[f]
        y = jnp.maximum(y, 0.0)
    y_ref[...] = y


def _dense0_call(hsc_t, w0s, tb, fw, fb):
    return pl.pallas_call(
        _dense0_body,
        out_shape=jax.ShapeDtypeStruct((NPAD, 128), jnp.float32),
    )(hsc_t, w0s, tb, fw, fb)


_BS = 2048


def _dense_body(nfc, xn_ref, ph_ref, wt_ref, tb_ref, fw_ref, fb_ref, y_ref):
    y = jnp.dot(xn_ref[...], wt_ref[0], preferred_element_type=jnp.float32, precision=lax.Precision.HIGHEST)
    for kk in range(1, 4):
        hk = ph_ref[kk - 1, 0] + ph_ref[kk - 1, 1]
        y = y + jnp.dot(hk, wt_ref[kk], preferred_element_type=jnp.float32, precision=lax.Precision.HIGHEST)
    y = y + tb_ref[...]
    y = jnp.maximum(y, 0.0)
    for f in range(nfc):
        y = jnp.dot(y, fw_ref[f], preferred_element_type=jnp.float32, precision=lax.Precision.HIGHEST) + fb_ref[f]
        y = jnp.maximum(y, 0.0)
    y_ref[...] = y


def _dense_call(xn, ph, wt, tb, fw, fb):
    nfc = fw.shape[0]
    return pl.pallas_call(
        functools.partial(_dense_body, nfc),
        grid=(NPAD // _BS,),
        in_specs=[
            pl.BlockSpec((_BS, 128), lambda i: (i, 0)),
            pl.BlockSpec((3, 2, _BS, 128), lambda i: (0, 0, i, 0)),
            pl.BlockSpec((4, 128, 128), lambda i: (0, 0, 0)),
            pl.BlockSpec((1, 128), lambda i: (0, 0)),
            pl.BlockSpec((nfc, 128, 128), lambda i: (0, 0, 0)),
            pl.BlockSpec((nfc, 1, 128), lambda i: (0, 0, 0)),
        ],
        out_specs=pl.BlockSpec((_BS, 128), lambda i: (i, 0)),
        out_shape=jax.ShapeDtypeStruct((NPAD, 128), jnp.float32),
    )(xn, ph, wt, tb, fw, fb)


def _topk_body(k_ref, s_ref, o_ref):
    kk = k_ref[0]
    idx = lax.broadcasted_iota(jnp.int32, (80, 128), 0) * 128 + \
        lax.broadcasted_iota(jnp.int32, (80, 128), 1)
    s = jnp.where(idx < N, s_ref[...], jnp.int32(-(2 ** 31)))

    def t_step(i, t):
        cand = t | (jnp.int32(1) << (30 - i))
        cnt = jnp.sum((s >= cand).astype(jnp.int32))
        return jnp.where(cnt >= kk, cand, t)

    T = lax.fori_loop(0, 31, t_step, jnp.int32(0))
    cnt_gt = jnp.sum((s > T).astype(jnp.int32))
    need = kk - cnt_gt
    eq = s == T

    def m_step(i, m):
        cand = m | (jnp.int32(1) << (13 - i))
        cnt = jnp.sum((eq & (idx < cand)).astype(jnp.int32))
        return jnp.where(cnt < need, cand, m)

    m = lax.fori_loop(0, 14, m_step, jnp.int32(0))
    mask = (s > T) | (eq & (idx <= m))
    o_ref[...] = mask.astype(jnp.float32)


def _topk_call(scores, k):
    s = scores + 0.0
    sbits = lax.bitcast_convert_type(s, jnp.int32).reshape(80, 128)
    karr = jnp.asarray(k, jnp.int32).reshape(1)
    out = pl.pallas_call(
        _topk_body,
        out_shape=jax.ShapeDtypeStruct((80, 128), jnp.float32),
        in_specs=[
            pl.BlockSpec(memory_space=pltpu.SMEM),
            pl.BlockSpec(memory_space=pltpu.VMEM),
        ],
        out_specs=pl.BlockSpec(memory_space=pltpu.VMEM),
    )(karr, sbits)
    return out.reshape(-1)[:N]


def kernel(x, edge_index, edge_attr, k, params):
    row = edge_index[0].astype(jnp.int32)
    col = edge_index[1].astype(jnp.int32)
    npadE = EPAD - E
    padi = (jnp.arange(npadE, dtype=jnp.int32) % N)
    rowp = jnp.concatenate([row, padi])
    colp = jnp.concatenate([col, padi])
    eap = jnp.concatenate([edge_attr, jnp.zeros((npadE,), jnp.float32)])
    rc = jnp.stack([rowp.reshape(NBLK, 16, 128), colp.reshape(NBLK, 16, 128)],
                   axis=1)
    eab = eap.reshape(NBLK, 16, 128)
    xpad = jnp.concatenate([x.astype(jnp.float32),
                            jnp.zeros((NPAD - N,), jnp.float32)])

    dg = _deg_call(rc, eab)
    dinv, xn0 = _prep_call(xpad, dg)
    wn, hsc = _nsc_call(rc, eab, dinv, xn0)

    p0 = params[0]
    w0s = jnp.concatenate([w for w in p0["tag_ws"]], axis=0)
    fw0 = jnp.stack([w for (w, _) in p0["fc"]])
    fb0 = jnp.stack([b.reshape(1, -1) for (_, b) in p0["fc"]])
    y = _dense0_call(hsc.T, w0s, p0["tag_b"].reshape(1, 128), fw0, fb0)

    for i in range(1, 6):
        p = params[i]
        xn = _norm_call(y)
        ph = _hops_call(xn, rc, wn)
        wt = jnp.stack(p["tag_ws"])
        tb = p["tag_b"].reshape(1, 128)
        if i < 5:
            fw = jnp.stack([w for (w, _) in p["fc"]])
            fb = jnp.stack([b.reshape(1, -1) for (_, b) in p["fc"]])
            y = _dense_call(xn, ph, wt, tb, fw, fb)
        else:
            w5 = jnp.zeros((128, 128), jnp.float32).at[:, 0:1].set(p["fc"][4][0])
            b5 = jnp.zeros((1, 128), jnp.float32).at[0, 0].set(p["fc"][4][1][0])
            fw = jnp.stack([w for (w, _) in p["fc"][:4]] + [w5])
            fb = jnp.stack([b.reshape(1, -1) for (_, b) in p["fc"][:4]] + [b5])
            y = _dense_call(xn, ph, wt, tb, fw, fb)
            scores = y[:, 0]

    mask = _topk_call(scores, k)
    return (mask, edge_attr)
